# baseline (speedup 1.0000x reference)
import functools

import jax
import jax.numpy as jnp
from jax import lax
from jax.experimental import pallas as pl
from jax.experimental.pallas import tpu as pltpu, tpu_sc as plsc

_CHUNK_ROWS = 16


def kernel(x, emb):
    n = x.shape[1]
    d = emb.shape[1]
    info = plsc.get_sparse_core_info()
    nc, ns = info.num_cores, info.num_subcores
    nw = nc * ns
    rows_w = n // nw
    n_ch = rows_w // _CHUNK_ROWS
    mesh = plsc.VectorSubcoreMesh(core_axis_name="c", subcore_axis_name="s")

    @functools.partial(
        pl.kernel,
        mesh=mesh,
        out_type=jax.ShapeDtypeStruct((n, d), emb.dtype),
        scratch_types=[
            pltpu.VMEM((2, _CHUNK_ROWS, d), emb.dtype),
            pltpu.SemaphoreType.DMA((2,)),
        ],
    )
    def run(emb_hbm, out_hbm, buf, osem):
        wid = lax.axis_index("s") * nc + lax.axis_index("c")
        base = wid * rows_w

        def out_copy(i, b):
            return pltpu.make_async_copy(
                buf.at[b],
                out_hbm.at[pl.ds(base + i * _CHUNK_ROWS, _CHUNK_ROWS), :],
                osem.at[b],
            )

        for i in range(n_ch):
            b = i % 2
            if i >= 2:
                out_copy(i - 2, b).wait()
            pltpu.sync_copy(
                emb_hbm.at[pl.ds(base + i * _CHUNK_ROWS, _CHUNK_ROWS), :],
                buf.at[b],
            )
            out_copy(i, b).start()
        for i in range(max(0, n_ch - 2), n_ch):
            out_copy(i, i % 2).wait()

    return run(emb)


# speedup vs baseline: 1.0688x; 1.0688x over previous
import functools

import jax
import jax.numpy as jnp
from jax import lax
from jax.experimental import pallas as pl
from jax.experimental.pallas import tpu as pltpu, tpu_sc as plsc

_CHUNK_ROWS = 32


def kernel(x, emb):
    n = x.shape[1]
    d = emb.shape[1]
    info = plsc.get_sparse_core_info()
    nc, ns = info.num_cores, info.num_subcores
    nw = nc * ns
    rows_w = n // nw
    n_ch = rows_w // _CHUNK_ROWS
    mesh = plsc.VectorSubcoreMesh(core_axis_name="c", subcore_axis_name="s")

    @functools.partial(
        pl.kernel,
        mesh=mesh,
        out_type=jax.ShapeDtypeStruct((n, d), emb.dtype),
        scratch_types=[
            pltpu.VMEM((2, _CHUNK_ROWS, d), emb.dtype),
            pltpu.SemaphoreType.DMA((2,)),
            pltpu.SemaphoreType.DMA((2,)),
        ],
    )
    def run(emb_hbm, out_hbm, buf, isem, osem):
        wid = lax.axis_index("s") * nc + lax.axis_index("c")
        base = wid * rows_w

        def in_copy(i, b):
            return pltpu.make_async_copy(
                emb_hbm.at[pl.ds(base + i * _CHUNK_ROWS, _CHUNK_ROWS), :],
                buf.at[b],
                isem.at[b],
            )

        def out_copy(i, b):
            return pltpu.make_async_copy(
                buf.at[b],
                out_hbm.at[pl.ds(base + i * _CHUNK_ROWS, _CHUNK_ROWS), :],
                osem.at[b],
            )

        in_copy(0, 0).start()
        for i in range(n_ch):
            b = i % 2
            in_copy(i, b).wait()
            out_copy(i, b).start()
            if i + 1 < n_ch:
                nb = (i + 1) % 2
                if i >= 1:
                    out_copy(i - 1, nb).wait()
                in_copy(i + 1, nb).start()
        for i in range(max(0, n_ch - 2), n_ch):
            out_copy(i, i % 2).wait()

    return run(emb)
